# Initial kernel scaffold; baseline (speedup 1.0000x reference)
import jax
import jax.numpy as jnp
from jax.experimental import pallas as pl


def kernel(x, edge_index, W1, b1, W2, b2, Wlin, blin):
    raise NotImplementedError("write your pallas kernel here")


# speedup vs baseline: 14.8404x; 14.8404x over previous
import functools

import jax
import jax.numpy as jnp
from jax import lax
from jax.experimental import pallas as pl
from jax.experimental.pallas import tpu as pltpu
from jax.experimental.pallas import tpu_sc as plsc

_N = 10000
_E = 320000
_D = 128
_NC = 2
_NS = 16
_NW = _NC * _NS
_EPW = _E // _NW
_K = 128
_FULL = _EPW // _K
_TAIL = _EPW - _FULL * _K
_ZR = 200
_NZ = _N // _ZR

def _round_robin(s, body):
  for j in range((_NZ + _NS - 1) // _NS):
    zi = s + _NS * j

    @pl.when(zi < _NZ)
    def _():
      body(zi * _ZR)


def _sc_scatter_body(g_hbm, src_hbm, dst_hbm, zrows_hbm, out_hbm,
                     src_v, dst_v, src_t, dst_t, rows_v, zbuf_v, agg_s, sem):
  c = lax.axis_index("c")
  s = lax.axis_index("s")
  wid = c * _NS + s

  pltpu.sync_copy(zrows_hbm, zbuf_v)
  _round_robin(s, lambda r: pltpu.sync_copy(zbuf_v, agg_s.at[pl.ds(r, _ZR)]))
  plsc.subcore_barrier()

  ebase = wid * _EPW

  def chunk(i, carry):
    base = ebase + i * _K
    pltpu.sync_copy(src_hbm.at[pl.ds(base, _K)], src_v)
    pltpu.sync_copy(dst_hbm.at[pl.ds(base, _K)], dst_v)
    pltpu.async_copy(g_hbm.at[src_v], rows_v, sem).wait()
    pltpu.sync_copy(rows_v, agg_s.at[dst_v], add=True)
    return carry

  lax.fori_loop(0, _FULL, chunk, 0)

  tbase = ebase + _FULL * _K
  pltpu.sync_copy(src_hbm.at[pl.ds(tbase, _TAIL)], src_t)
  pltpu.sync_copy(dst_hbm.at[pl.ds(tbase, _TAIL)], dst_t)
  pltpu.async_copy(g_hbm.at[src_t], rows_v.at[pl.ds(0, _TAIL)], sem).wait()
  pltpu.sync_copy(rows_v.at[pl.ds(0, _TAIL)], agg_s.at[dst_t], add=True)

  plsc.subcore_barrier()
  _round_robin(s, lambda r: pltpu.sync_copy(
      agg_s.at[pl.ds(r, _ZR)], out_hbm.at[pl.ds(c * _N + r, _ZR)]))


@functools.cache
def _get_sc_scatter():
  mesh = plsc.VectorSubcoreMesh(core_axis_name="c", subcore_axis_name="s",
                                num_cores=_NC, num_subcores=_NS)
  return pl.kernel(
      _sc_scatter_body,
      out_type=jax.ShapeDtypeStruct((_NC * _N, _D), jnp.float32),
      mesh=mesh,
      scratch_types=[
          pltpu.VMEM((_K,), jnp.int32),
          pltpu.VMEM((_K,), jnp.int32),
          pltpu.VMEM((_TAIL,), jnp.int32),
          pltpu.VMEM((_TAIL,), jnp.int32),
          pltpu.VMEM((_K, _D), jnp.float32),
          pltpu.VMEM((_ZR, _D), jnp.float32),
          pltpu.VMEM_SHARED((_N, _D), jnp.float32),
          pltpu.SemaphoreType.DMA,
      ],
  )


def _sc_deg_body(dst_hbm, ones_hbm, zrows_hbm, out_hbm,
                 dst_v, dst_t, ones_v, z_v, deg_s):
  c = lax.axis_index("c")
  s = lax.axis_index("s")
  wid = c * _NS + s

  pltpu.sync_copy(zrows_hbm, z_v)
  _round_robin(s, lambda r: pltpu.sync_copy(z_v, deg_s.at[pl.ds(r, _ZR)]))
  pltpu.sync_copy(ones_hbm, ones_v)
  plsc.subcore_barrier()

  ebase = wid * _EPW

  def chunk(i, carry):
    pltpu.sync_copy(dst_hbm.at[pl.ds(ebase + i * _K, _K)], dst_v)
    pltpu.sync_copy(ones_v, deg_s.at[dst_v], add=True)
    return carry

  lax.fori_loop(0, _FULL, chunk, 0)
  pltpu.sync_copy(dst_hbm.at[pl.ds(ebase + _FULL * _K, _TAIL)], dst_t)
  pltpu.sync_copy(ones_v.at[pl.ds(0, _TAIL)], deg_s.at[dst_t], add=True)

  plsc.subcore_barrier()
  _round_robin(s, lambda r: pltpu.sync_copy(
      deg_s.at[pl.ds(r, _ZR)], out_hbm.at[pl.ds(c * _N + r, _ZR)]))


@functools.cache
def _get_sc_deg():
  mesh = plsc.VectorSubcoreMesh(core_axis_name="c", subcore_axis_name="s",
                                num_cores=_NC, num_subcores=_NS)
  return pl.kernel(
      _sc_deg_body,
      out_type=jax.ShapeDtypeStruct((_NC * _N, _D), jnp.float32),
      mesh=mesh,
      scratch_types=[
          pltpu.VMEM((_K,), jnp.int32),
          pltpu.VMEM((_TAIL,), jnp.int32),
          pltpu.VMEM((_K, _D), jnp.float32),
          pltpu.VMEM((_ZR, _D), jnp.float32),
          pltpu.VMEM_SHARED((_N, _D), jnp.float32),
      ],
  )

_R = 1000
_G = _N // _R


def _tc1_body(dlo_ref, dhi_ref, x_ref, w_ref, g_ref, dinv_ref):
  deg = dlo_ref[:, 0:1] + dhi_ref[:, 0:1] + 1.0
  dinv = lax.rsqrt(deg)
  h = jnp.dot(x_ref[...], w_ref[...], preferred_element_type=jnp.float32)
  g_ref[...] = h * dinv
  dinv_ref[...] = dinv


_tc1 = pl.pallas_call(
    _tc1_body,
    grid=(_G,),
    in_specs=[
        pl.BlockSpec((_R, _D), lambda i: (i, 0)),
        pl.BlockSpec((_R, _D), lambda i: (i + _G, 0)),
        pl.BlockSpec((_R, _D), lambda i: (i, 0)),
        pl.BlockSpec((_D, _D), lambda i: (0, 0)),
    ],
    out_specs=[
        pl.BlockSpec((_R, _D), lambda i: (i, 0)),
        pl.BlockSpec((_R, 1), lambda i: (i, 0)),
    ],
    out_shape=[
        jax.ShapeDtypeStruct((_N, _D), jnp.float32),
        jax.ShapeDtypeStruct((_N, 1), jnp.float32),
    ],
)


def _tc2_body(alo_ref, ahi_ref, g_ref, dinv_ref, b_ref, w_ref, o_ref):
  dinv = dinv_ref[...]
  pre = (alo_ref[...] + ahi_ref[...] + g_ref[...]) * dinv + b_ref[...]
  x2 = jnp.maximum(pre, 0.0)
  o_ref[...] = jnp.dot(
      x2, w_ref[...], preferred_element_type=jnp.float32) * dinv


_tc2 = pl.pallas_call(
    _tc2_body,
    grid=(_G,),
    in_specs=[
        pl.BlockSpec((_R, _D), lambda i: (i, 0)),
        pl.BlockSpec((_R, _D), lambda i: (i + _G, 0)),
        pl.BlockSpec((_R, _D), lambda i: (i, 0)),
        pl.BlockSpec((_R, 1), lambda i: (i, 0)),
        pl.BlockSpec((1, _D), lambda i: (0, 0)),
        pl.BlockSpec((_D, _D), lambda i: (0, 0)),
    ],
    out_specs=pl.BlockSpec((_R, _D), lambda i: (i, 0)),
    out_shape=jax.ShapeDtypeStruct((_N, _D), jnp.float32),
)

_C = 40


def _tc3_body(alo_ref, ahi_ref, g_ref, dinv_ref, b_ref, wl_ref, bl_ref, o_ref):
  pre = (alo_ref[...] + ahi_ref[...] + g_ref[...]) * dinv_ref[...] + b_ref[...]
  x3 = jnp.maximum(pre, 0.0)
  o_ref[...] = jnp.dot(
      x3, wl_ref[...], preferred_element_type=jnp.float32) + bl_ref[...]


_tc3 = pl.pallas_call(
    _tc3_body,
    grid=(_G,),
    in_specs=[
        pl.BlockSpec((_R, _D), lambda i: (i, 0)),
        pl.BlockSpec((_R, _D), lambda i: (i + _G, 0)),
        pl.BlockSpec((_R, _D), lambda i: (i, 0)),
        pl.BlockSpec((_R, 1), lambda i: (i, 0)),
        pl.BlockSpec((1, _D), lambda i: (0, 0)),
        pl.BlockSpec((_D, _C), lambda i: (0, 0)),
        pl.BlockSpec((1, _C), lambda i: (0, 0)),
    ],
    out_specs=pl.BlockSpec((_R, _C), lambda i: (i, 0)),
    out_shape=jax.ShapeDtypeStruct((_N, _C), jnp.float32),
)


def kernel(x, edge_index, W1, b1, W2, b2, Wlin, blin):
  src = edge_index[0]
  dst = edge_index[1]
  zrows = jnp.zeros((_ZR, _D), jnp.float32)
  ones = jnp.ones((_K, _D), jnp.float32)

  sc_deg = _get_sc_deg()
  sc_scatter = _get_sc_scatter()
  degp = sc_deg(dst, ones, zrows)
  g1, dinv = _tc1(degp, degp, x, W1)
  a1 = sc_scatter(g1, src, dst, zrows)
  g2 = _tc2(a1, a1, g1, dinv, b1.reshape(1, _D), W2)
  a2 = sc_scatter(g2, src, dst, zrows)
  return _tc3(a2, a2, g2, dinv, b2.reshape(1, _D), Wlin, blin.reshape(1, _C))


# speedup vs baseline: 25.8664x; 1.7430x over previous
import functools

import jax
import jax.numpy as jnp
from jax import lax
from jax.experimental import pallas as pl
from jax.experimental.pallas import tpu as pltpu
from jax.experimental.pallas import tpu_sc as plsc

_N = 10000
_E = 320000
_D = 128
_NC = 2
_NS = 16
_NW = _NC * _NS
_K = 128
_CHUNKS = _E // _K
_FPW = _CHUNKS // _NW
_NB = 3
_ZR = 200
_NZ = _N // _ZR

def _round_robin(s, body):
  for j in range((_NZ + _NS - 1) // _NS):
    zi = s + _NS * j

    @pl.when(zi < _NZ)
    def _():
      body(zi * _ZR)


_ZF = _N // _K


def _zero_rows(s, zsrc, dstref):
  for j in range((_ZF + _NS - 1) // _NS):
    zi = s + _NS * j

    @pl.when(zi < _ZF)
    def _():
      pltpu.sync_copy(zsrc, dstref.at[pl.ds(zi * _K, _K)])

  @pl.when(s == 0)
  def _():
    pltpu.sync_copy(zsrc.at[pl.ds(0, _N - _ZF * _K)],
                    dstref.at[pl.ds(_ZF * _K, _N - _ZF * _K)])


def _sc_scatter_body(g_hbm, src_hbm, dst_hbm, zk_hbm, out_hbm,
                     src0, src1, src2, dst0, dst1, dst2, row0, row1, row2,
                     agg_s, gs0, gs1, gs2, ss0, ss1, ss2):
  srcs, dsts, rows = [src0, src1, src2], [dst0, dst1, dst2], [row0, row1, row2]
  gsem, ssem = [gs0, gs1, gs2], [ss0, ss1, ss2]
  c = lax.axis_index("c")
  s = lax.axis_index("s")
  wid = c * _NS + s
  cnt = _FPW + jnp.where(wid < _CHUNKS - _FPW * _NW, 1, 0)

  pltpu.sync_copy(zk_hbm, row0)
  _zero_rows(s, row0, agg_s)
  plsc.subcore_barrier()

  def fire_gather(i, b):
    base = _K * wid + (_K * _NW) * i
    pltpu.sync_copy(src_hbm.at[pl.ds(base, _K)], srcs[b])
    pltpu.sync_copy(dst_hbm.at[pl.ds(base, _K)], dsts[b])
    pltpu.async_copy(g_hbm.at[srcs[b]], rows[b], gsem[b])

  def wait_scatter(b):
    pltpu.make_async_copy(rows[b], agg_s.at[pl.ds(0, _K)], ssem[b]).wait()

  fire_gather(0, 0)

  def group(it, carry):
    for b in range(_NB):
      i = _NB * it + b
      nb = (b + 1) % _NB

      @pl.when(jnp.logical_and(i + 1 < cnt, i >= _NB - 1))
      def _():
        wait_scatter(nb)

      @pl.when(i + 1 < cnt)
      def _():
        fire_gather(i + 1, nb)

      @pl.when(i < cnt)
      def _():
        pltpu.make_async_copy(g_hbm.at[pl.ds(0, _K)], rows[b], gsem[b]).wait()
        pltpu.async_copy(rows[b], agg_s.at[dsts[b]], ssem[b], add=True)
    return carry

  lax.fori_loop(0, (_FPW + 1 + _NB - 1) // _NB, group, 0)
  for b in range(_NB):
    wait_scatter(b)

  plsc.subcore_barrier()
  _round_robin(s, lambda r: pltpu.sync_copy(
      agg_s.at[pl.ds(r, _ZR)], out_hbm.at[pl.ds(c * _N + r, _ZR)]))


@functools.cache
def _get_sc_scatter():
  mesh = plsc.VectorSubcoreMesh(core_axis_name="c", subcore_axis_name="s",
                                num_cores=_NC, num_subcores=_NS)
  return pl.kernel(
      _sc_scatter_body,
      out_type=jax.ShapeDtypeStruct((_NC * _N, _D), jnp.float32),
      mesh=mesh,
      scratch_types=(
          [pltpu.VMEM((_K,), jnp.int32)] * 6
          + [pltpu.VMEM((_K, _D), jnp.float32)] * 3
          + [pltpu.VMEM_SHARED((_N, _D), jnp.float32)]
          + [pltpu.SemaphoreType.DMA] * 6
      ),
  )


def _sc_deg_body(dst_hbm, ones_hbm, zk_hbm, out_hbm,
                 dst0, dst1, dst2, ones_v, z_v, deg_s, ss0, ss1, ss2):
  dsts, ssem = [dst0, dst1, dst2], [ss0, ss1, ss2]
  c = lax.axis_index("c")
  s = lax.axis_index("s")
  wid = c * _NS + s
  cnt = _FPW + jnp.where(wid < _CHUNKS - _FPW * _NW, 1, 0)

  pltpu.sync_copy(zk_hbm, z_v)
  _zero_rows(s, z_v, deg_s)
  pltpu.sync_copy(ones_hbm, ones_v)
  plsc.subcore_barrier()

  def load_idx(i, b):
    base = _K * wid + (_K * _NW) * i
    pltpu.sync_copy(dst_hbm.at[pl.ds(base, _K)], dsts[b])

  def wait_scatter(b):
    pltpu.make_async_copy(ones_v, deg_s.at[pl.ds(0, _K)], ssem[b]).wait()

  load_idx(0, 0)

  def group(it, carry):
    for b in range(_NB):
      i = _NB * it + b
      nb = (b + 1) % _NB

      @pl.when(jnp.logical_and(i + 1 < cnt, i >= _NB - 1))
      def _():
        wait_scatter(nb)

      @pl.when(i + 1 < cnt)
      def _():
        load_idx(i + 1, nb)

      @pl.when(i < cnt)
      def _():
        pltpu.async_copy(ones_v, deg_s.at[dsts[b]], ssem[b], add=True)
    return carry

  lax.fori_loop(0, (_FPW + 1 + _NB - 1) // _NB, group, 0)
  for b in range(_NB):
    wait_scatter(b)

  plsc.subcore_barrier()
  _round_robin(s, lambda r: pltpu.sync_copy(
      deg_s.at[pl.ds(r, _ZR)], out_hbm.at[pl.ds(c * _N + r, _ZR)]))


@functools.cache
def _get_sc_deg():
  mesh = plsc.VectorSubcoreMesh(core_axis_name="c", subcore_axis_name="s",
                                num_cores=_NC, num_subcores=_NS)
  return pl.kernel(
      _sc_deg_body,
      out_type=jax.ShapeDtypeStruct((_NC * _N, _D), jnp.float32),
      mesh=mesh,
      scratch_types=(
          [pltpu.VMEM((_K,), jnp.int32)] * 3
          + [pltpu.VMEM((_K, _D), jnp.float32),
             pltpu.VMEM((_K, _D), jnp.float32),
             pltpu.VMEM_SHARED((_N, _D), jnp.float32)]
          + [pltpu.SemaphoreType.DMA] * 3
      ),
  )

_R = 1000
_G = _N // _R


def _tc1_body(dlo_ref, dhi_ref, x_ref, w_ref, g_ref, dinv_ref):
  deg = dlo_ref[:, 0:1] + dhi_ref[:, 0:1] + 1.0
  dinv = lax.rsqrt(deg)
  h = jnp.dot(x_ref[...], w_ref[...], preferred_element_type=jnp.float32)
  g_ref[...] = h * dinv
  dinv_ref[...] = dinv


_tc1 = pl.pallas_call(
    _tc1_body,
    grid=(_G,),
    in_specs=[
        pl.BlockSpec((_R, _D), lambda i: (i, 0)),
        pl.BlockSpec((_R, _D), lambda i: (i + _G, 0)),
        pl.BlockSpec((_R, _D), lambda i: (i, 0)),
        pl.BlockSpec((_D, _D), lambda i: (0, 0)),
    ],
    out_specs=[
        pl.BlockSpec((_R, _D), lambda i: (i, 0)),
        pl.BlockSpec((_R, 1), lambda i: (i, 0)),
    ],
    out_shape=[
        jax.ShapeDtypeStruct((_N, _D), jnp.float32),
        jax.ShapeDtypeStruct((_N, 1), jnp.float32),
    ],
)


def _tc2_body(alo_ref, ahi_ref, g_ref, dinv_ref, b_ref, w_ref, o_ref):
  dinv = dinv_ref[...]
  pre = (alo_ref[...] + ahi_ref[...] + g_ref[...]) * dinv + b_ref[...]
  x2 = jnp.maximum(pre, 0.0)
  o_ref[...] = jnp.dot(
      x2, w_ref[...], preferred_element_type=jnp.float32) * dinv


_tc2 = pl.pallas_call(
    _tc2_body,
    grid=(_G,),
    in_specs=[
        pl.BlockSpec((_R, _D), lambda i: (i, 0)),
        pl.BlockSpec((_R, _D), lambda i: (i + _G, 0)),
        pl.BlockSpec((_R, _D), lambda i: (i, 0)),
        pl.BlockSpec((_R, 1), lambda i: (i, 0)),
        pl.BlockSpec((1, _D), lambda i: (0, 0)),
        pl.BlockSpec((_D, _D), lambda i: (0, 0)),
    ],
    out_specs=pl.BlockSpec((_R, _D), lambda i: (i, 0)),
    out_shape=jax.ShapeDtypeStruct((_N, _D), jnp.float32),
)

_C = 40


def _tc3_body(alo_ref, ahi_ref, g_ref, dinv_ref, b_ref, wl_ref, bl_ref, o_ref):
  pre = (alo_ref[...] + ahi_ref[...] + g_ref[...]) * dinv_ref[...] + b_ref[...]
  x3 = jnp.maximum(pre, 0.0)
  o_ref[...] = jnp.dot(
      x3, wl_ref[...], preferred_element_type=jnp.float32) + bl_ref[...]


_tc3 = pl.pallas_call(
    _tc3_body,
    grid=(_G,),
    in_specs=[
        pl.BlockSpec((_R, _D), lambda i: (i, 0)),
        pl.BlockSpec((_R, _D), lambda i: (i + _G, 0)),
        pl.BlockSpec((_R, _D), lambda i: (i, 0)),
        pl.BlockSpec((_R, 1), lambda i: (i, 0)),
        pl.BlockSpec((1, _D), lambda i: (0, 0)),
        pl.BlockSpec((_D, _C), lambda i: (0, 0)),
        pl.BlockSpec((1, _C), lambda i: (0, 0)),
    ],
    out_specs=pl.BlockSpec((_R, _C), lambda i: (i, 0)),
    out_shape=jax.ShapeDtypeStruct((_N, _C), jnp.float32),
)


def kernel(x, edge_index, W1, b1, W2, b2, Wlin, blin):
  src = edge_index[0]
  dst = edge_index[1]
  zk = jnp.zeros((_K, _D), jnp.float32)
  ones = jnp.ones((_K, _D), jnp.float32)

  sc_deg = _get_sc_deg()
  sc_scatter = _get_sc_scatter()
  degp = sc_deg(dst, ones, zk)
  g1, dinv = _tc1(degp, degp, x, W1)
  a1 = sc_scatter(g1, src, dst, zk)
  g2 = _tc2(a1, a1, g1, dinv, b1.reshape(1, _D), W2)
  a2 = sc_scatter(g2, src, dst, zk)
  return _tc3(a2, a2, g2, dinv, b2.reshape(1, _D), Wlin, blin.reshape(1, _C))


# speedup vs baseline: 27.8352x; 1.0761x over previous
import functools

import jax
import jax.numpy as jnp
from jax import lax
from jax.experimental import pallas as pl
from jax.experimental.pallas import tpu as pltpu
from jax.experimental.pallas import tpu_sc as plsc

_N = 10000
_E = 320000
_D = 128
_NC = 2
_NS = 16
_NW = _NC * _NS
_K = 128
_CHUNKS = _E // _K
_FPW = _CHUNKS // _NW
_NB = 3
_ZR = 200
_NZ = _N // _ZR

def _round_robin(s, body):
  for j in range((_NZ + _NS - 1) // _NS):
    zi = s + _NS * j

    @pl.when(zi < _NZ)
    def _():
      body(zi * _ZR)


_ZF = _N // _K


def _zero_rows(s, zsrc, dstref):
  for j in range((_ZF + _NS - 1) // _NS):
    zi = s + _NS * j

    @pl.when(zi < _ZF)
    def _():
      pltpu.sync_copy(zsrc, dstref.at[pl.ds(zi * _K, _K)])

  @pl.when(s == 0)
  def _():
    pltpu.sync_copy(zsrc.at[pl.ds(0, _N - _ZF * _K)],
                    dstref.at[pl.ds(_ZF * _K, _N - _ZF * _K)])


def _sc_scatter_body(g_hbm, src_hbm, dst_hbm, zk_hbm, out_hbm,
                     src0, src1, src2, src3, dst0, dst1, dst2, dst3,
                     row0, row1, agg_s,
                     is0, is1, is2, is3, gs0, gs1, ss0, ss1):
  srcs, dsts = [src0, src1, src2, src3], [dst0, dst1, dst2, dst3]
  rows = [row0, row1]
  isem, gsem, ssem = [is0, is1, is2, is3], [gs0, gs1], [ss0, ss1]
  c = lax.axis_index("c")
  s = lax.axis_index("s")
  wid = c * _NS + s
  cnt = _FPW + jnp.where(wid < _CHUNKS - _FPW * _NW, 1, 0)

  pltpu.sync_copy(zk_hbm, row0)
  _zero_rows(s, row0, agg_s)
  plsc.subcore_barrier()

  def fire_idx(i, b):
    base = _K * wid + (_K * _NW) * i
    pltpu.async_copy(src_hbm.at[pl.ds(base, _K)], srcs[b], isem[b])
    pltpu.async_copy(dst_hbm.at[pl.ds(base, _K)], dsts[b], isem[b])

  def wait_idx(b):
    pltpu.make_async_copy(src_hbm.at[pl.ds(0, _K)], srcs[b], isem[b]).wait()
    pltpu.make_async_copy(dst_hbm.at[pl.ds(0, _K)], dsts[b], isem[b]).wait()

  def wait_scatter(b):
    pltpu.make_async_copy(rows[b], agg_s.at[pl.ds(0, _K)], ssem[b]).wait()

  fire_idx(0, 0)
  fire_idx(1, 1)
  wait_idx(0)
  pltpu.async_copy(g_hbm.at[srcs[0]], rows[0], gsem[0])

  def group(it, carry):
    for t in range(4):
      i = 4 * it + t
      r, nr = t % 2, (t + 1) % 2
      nb = (t + 1) % 4

      @pl.when(jnp.logical_and(i >= 1, i + 1 < cnt))
      def _():
        wait_scatter(nr)

      @pl.when(i + 2 < cnt)
      def _():
        fire_idx(i + 2, (t + 2) % 4)

      @pl.when(i + 1 < cnt)
      def _():
        wait_idx(nb)
        pltpu.async_copy(g_hbm.at[srcs[nb]], rows[nr], gsem[nr])

      @pl.when(i < cnt)
      def _():
        pltpu.make_async_copy(g_hbm.at[pl.ds(0, _K)], rows[r], gsem[r]).wait()
        pltpu.async_copy(rows[r], agg_s.at[dsts[t]], ssem[r], add=True)
    return carry

  lax.fori_loop(0, (_FPW + 1 + 3) // 4, group, 0)
  wait_scatter(0)
  wait_scatter(1)

  plsc.subcore_barrier()
  _round_robin(s, lambda r: pltpu.sync_copy(
      agg_s.at[pl.ds(r, _ZR)], out_hbm.at[pl.ds(c * _N + r, _ZR)]))


@functools.cache
def _get_sc_scatter():
  mesh = plsc.VectorSubcoreMesh(core_axis_name="c", subcore_axis_name="s",
                                num_cores=_NC, num_subcores=_NS)
  return pl.kernel(
      _sc_scatter_body,
      out_type=jax.ShapeDtypeStruct((_NC * _N, _D), jnp.float32),
      mesh=mesh,
      scratch_types=(
          [pltpu.VMEM((_K,), jnp.int32)] * 8
          + [pltpu.VMEM((_K, _D), jnp.float32)] * 2
          + [pltpu.VMEM_SHARED((_N, _D), jnp.float32)]
          + [pltpu.SemaphoreType.DMA] * 8
      ),
  )


def _sc_deg_body(dst_hbm, ones_hbm, zk_hbm, out_hbm,
                 dst0, dst1, dst2, dst3, ones_v, z_v, deg_s,
                 is0, is1, is2, is3, ss0, ss1):
  dsts = [dst0, dst1, dst2, dst3]
  isem, ssem = [is0, is1, is2, is3], [ss0, ss1]
  c = lax.axis_index("c")
  s = lax.axis_index("s")
  wid = c * _NS + s
  cnt = _FPW + jnp.where(wid < _CHUNKS - _FPW * _NW, 1, 0)

  pltpu.sync_copy(zk_hbm, z_v)
  _zero_rows(s, z_v, deg_s)
  pltpu.sync_copy(ones_hbm, ones_v)
  plsc.subcore_barrier()

  def fire_idx(i, b):
    base = _K * wid + (_K * _NW) * i
    pltpu.async_copy(dst_hbm.at[pl.ds(base, _K)], dsts[b], isem[b])

  def wait_idx(b):
    pltpu.make_async_copy(dst_hbm.at[pl.ds(0, _K)], dsts[b], isem[b]).wait()

  def wait_scatter(b):
    pltpu.make_async_copy(ones_v, deg_s.at[pl.ds(0, _K)], ssem[b]).wait()

  fire_idx(0, 0)
  fire_idx(1, 1)
  wait_idx(0)

  def group(it, carry):
    for t in range(4):
      i = 4 * it + t
      r, nr = t % 2, (t + 1) % 2
      nb = (t + 1) % 4

      @pl.when(jnp.logical_and(i >= 1, i + 1 < cnt))
      def _():
        wait_scatter(nr)

      @pl.when(i + 2 < cnt)
      def _():
        fire_idx(i + 2, (t + 2) % 4)

      @pl.when(i + 1 < cnt)
      def _():
        wait_idx(nb)

      @pl.when(i < cnt)
      def _():
        pltpu.async_copy(ones_v, deg_s.at[dsts[t]], ssem[r], add=True)
    return carry

  lax.fori_loop(0, (_FPW + 1 + 3) // 4, group, 0)
  wait_scatter(0)
  wait_scatter(1)

  plsc.subcore_barrier()
  _round_robin(s, lambda r: pltpu.sync_copy(
      deg_s.at[pl.ds(r, _ZR)], out_hbm.at[pl.ds(c * _N + r, _ZR)]))


@functools.cache
def _get_sc_deg():
  mesh = plsc.VectorSubcoreMesh(core_axis_name="c", subcore_axis_name="s",
                                num_cores=_NC, num_subcores=_NS)
  return pl.kernel(
      _sc_deg_body,
      out_type=jax.ShapeDtypeStruct((_NC * _N, _D), jnp.float32),
      mesh=mesh,
      scratch_types=(
          [pltpu.VMEM((_K,), jnp.int32)] * 4
          + [pltpu.VMEM((_K, _D), jnp.float32),
             pltpu.VMEM((_K, _D), jnp.float32),
             pltpu.VMEM_SHARED((_N, _D), jnp.float32)]
          + [pltpu.SemaphoreType.DMA] * 6
      ),
  )

_R = 1000
_G = _N // _R


def _tc1_body(dlo_ref, dhi_ref, x_ref, w_ref, g_ref, dinv_ref):
  deg = dlo_ref[:, 0:1] + dhi_ref[:, 0:1] + 1.0
  dinv = lax.rsqrt(deg)
  h = jnp.dot(x_ref[...], w_ref[...], preferred_element_type=jnp.float32)
  g_ref[...] = h * dinv
  dinv_ref[...] = dinv


_tc1 = pl.pallas_call(
    _tc1_body,
    grid=(_G,),
    in_specs=[
        pl.BlockSpec((_R, _D), lambda i: (i, 0)),
        pl.BlockSpec((_R, _D), lambda i: (i + _G, 0)),
        pl.BlockSpec((_R, _D), lambda i: (i, 0)),
        pl.BlockSpec((_D, _D), lambda i: (0, 0)),
    ],
    out_specs=[
        pl.BlockSpec((_R, _D), lambda i: (i, 0)),
        pl.BlockSpec((_R, 1), lambda i: (i, 0)),
    ],
    out_shape=[
        jax.ShapeDtypeStruct((_N, _D), jnp.float32),
        jax.ShapeDtypeStruct((_N, 1), jnp.float32),
    ],
)


def _tc2_body(alo_ref, ahi_ref, g_ref, dinv_ref, b_ref, w_ref, o_ref):
  dinv = dinv_ref[...]
  pre = (alo_ref[...] + ahi_ref[...] + g_ref[...]) * dinv + b_ref[...]
  x2 = jnp.maximum(pre, 0.0)
  o_ref[...] = jnp.dot(
      x2, w_ref[...], preferred_element_type=jnp.float32) * dinv


_tc2 = pl.pallas_call(
    _tc2_body,
    grid=(_G,),
    in_specs=[
        pl.BlockSpec((_R, _D), lambda i: (i, 0)),
        pl.BlockSpec((_R, _D), lambda i: (i + _G, 0)),
        pl.BlockSpec((_R, _D), lambda i: (i, 0)),
        pl.BlockSpec((_R, 1), lambda i: (i, 0)),
        pl.BlockSpec((1, _D), lambda i: (0, 0)),
        pl.BlockSpec((_D, _D), lambda i: (0, 0)),
    ],
    out_specs=pl.BlockSpec((_R, _D), lambda i: (i, 0)),
    out_shape=jax.ShapeDtypeStruct((_N, _D), jnp.float32),
)

_C = 40


def _tc3_body(alo_ref, ahi_ref, g_ref, dinv_ref, b_ref, wl_ref, bl_ref, o_ref):
  pre = (alo_ref[...] + ahi_ref[...] + g_ref[...]) * dinv_ref[...] + b_ref[...]
  x3 = jnp.maximum(pre, 0.0)
  o_ref[...] = jnp.dot(
      x3, wl_ref[...], preferred_element_type=jnp.float32) + bl_ref[...]


_tc3 = pl.pallas_call(
    _tc3_body,
    grid=(_G,),
    in_specs=[
        pl.BlockSpec((_R, _D), lambda i: (i, 0)),
        pl.BlockSpec((_R, _D), lambda i: (i + _G, 0)),
        pl.BlockSpec((_R, _D), lambda i: (i, 0)),
        pl.BlockSpec((_R, 1), lambda i: (i, 0)),
        pl.BlockSpec((1, _D), lambda i: (0, 0)),
        pl.BlockSpec((_D, _C), lambda i: (0, 0)),
        pl.BlockSpec((1, _C), lambda i: (0, 0)),
    ],
    out_specs=pl.BlockSpec((_R, _C), lambda i: (i, 0)),
    out_shape=jax.ShapeDtypeStruct((_N, _C), jnp.float32),
)


def kernel(x, edge_index, W1, b1, W2, b2, Wlin, blin):
  src = edge_index[0]
  dst = edge_index[1]
  zk = jnp.zeros((_K, _D), jnp.float32)
  ones = jnp.ones((_K, _D), jnp.float32)

  sc_deg = _get_sc_deg()
  sc_scatter = _get_sc_scatter()
  degp = sc_deg(dst, ones, zk)
  g1, dinv = _tc1(degp, degp, x, W1)
  a1 = sc_scatter(g1, src, dst, zk)
  g2 = _tc2(a1, a1, g1, dinv, b1.reshape(1, _D), W2)
  a2 = sc_scatter(g2, src, dst, zk)
  return _tc3(a2, a2, g2, dinv, b2.reshape(1, _D), Wlin, blin.reshape(1, _C))


# speedup vs baseline: 28.6932x; 1.0308x over previous
import functools

import jax
import jax.numpy as jnp
from jax import lax
from jax.experimental import pallas as pl
from jax.experimental.pallas import tpu as pltpu
from jax.experimental.pallas import tpu_sc as plsc

_N = 10000
_E = 320000
_D = 128
_NC = 2
_NS = 16
_NW = _NC * _NS
_K = 128
_CHUNKS = _E // _K
_FPW = _CHUNKS // _NW
_NB = 3
_ZR = 200
_NZ = _N // _ZR

def _round_robin(s, body):
  for j in range((_NZ + _NS - 1) // _NS):
    zi = s + _NS * j

    @pl.when(zi < _NZ)
    def _():
      body(zi * _ZR)


_ZF = _N // _K


def _zero_rows(s, zsrc, dstref):
  for j in range((_ZF + _NS - 1) // _NS):
    zi = s + _NS * j

    @pl.when(zi < _ZF)
    def _():
      pltpu.sync_copy(zsrc, dstref.at[pl.ds(zi * _K, _K)])

  @pl.when(s == 0)
  def _():
    pltpu.sync_copy(zsrc.at[pl.ds(0, _N - _ZF * _K)],
                    dstref.at[pl.ds(_ZF * _K, _N - _ZF * _K)])


def _sc_scatter_body(g_hbm, src_hbm, dst_hbm, zk_hbm, out_hbm,
                     src0, src1, src2, src3, dst0, dst1, dst2, dst3,
                     row0, row1, row2, agg_s,
                     is0, is1, is2, is3, gs0, gs1, gs2, ss0, ss1, ss2):
  srcs, dsts = [src0, src1, src2, src3], [dst0, dst1, dst2, dst3]
  rows = [row0, row1, row2]
  isem, gsem, ssem = [is0, is1, is2, is3], [gs0, gs1, gs2], [ss0, ss1, ss2]
  c = lax.axis_index("c")
  s = lax.axis_index("s")
  wid = c * _NS + s
  cnt = _FPW + jnp.where(wid < _CHUNKS - _FPW * _NW, 1, 0)

  pltpu.sync_copy(zk_hbm, row0)
  _zero_rows(s, row0, agg_s)
  plsc.subcore_barrier()

  def fire_idx(i, b):
    base = _K * wid + (_K * _NW) * i
    pltpu.async_copy(src_hbm.at[pl.ds(base, _K)], srcs[b], isem[b])
    pltpu.async_copy(dst_hbm.at[pl.ds(base, _K)], dsts[b], isem[b])

  def wait_idx(b):
    pltpu.make_async_copy(src_hbm.at[pl.ds(0, _K)], srcs[b], isem[b]).wait()
    pltpu.make_async_copy(dst_hbm.at[pl.ds(0, _K)], dsts[b], isem[b]).wait()

  def wait_scatter(b):
    pltpu.make_async_copy(rows[b], agg_s.at[pl.ds(0, _K)], ssem[b]).wait()

  fire_idx(0, 0)
  fire_idx(1, 1)
  wait_idx(0)
  pltpu.async_copy(g_hbm.at[srcs[0]], rows[0], gsem[0])

  def group(it, carry):
    for t in range(12):
      i = 12 * it + t
      r, nr = t % 3, (t + 1) % 3
      nb = (t + 1) % 4

      @pl.when(jnp.logical_and(i >= 2, i + 1 < cnt))
      def _():
        wait_scatter(nr)

      @pl.when(i + 2 < cnt)
      def _():
        fire_idx(i + 2, (t + 2) % 4)

      @pl.when(i + 1 < cnt)
      def _():
        wait_idx(nb)
        pltpu.async_copy(g_hbm.at[srcs[nb]], rows[nr], gsem[nr])

      @pl.when(i < cnt)
      def _():
        pltpu.make_async_copy(g_hbm.at[pl.ds(0, _K)], rows[r], gsem[r]).wait()
        pltpu.async_copy(rows[r], agg_s.at[dsts[t % 4]], ssem[r], add=True)
    return carry

  lax.fori_loop(0, (_FPW + 1 + 11) // 12, group, 0)
  wait_scatter(0)
  wait_scatter(1)
  wait_scatter(2)

  plsc.subcore_barrier()
  _round_robin(s, lambda r: pltpu.sync_copy(
      agg_s.at[pl.ds(r, _ZR)], out_hbm.at[pl.ds(c * _N + r, _ZR)]))


@functools.cache
def _get_sc_scatter():
  mesh = plsc.VectorSubcoreMesh(core_axis_name="c", subcore_axis_name="s",
                                num_cores=_NC, num_subcores=_NS)
  return pl.kernel(
      _sc_scatter_body,
      out_type=jax.ShapeDtypeStruct((_NC * _N, _D), jnp.float32),
      mesh=mesh,
      scratch_types=(
          [pltpu.VMEM((_K,), jnp.int32)] * 8
          + [pltpu.VMEM((_K, _D), jnp.float32)] * 3
          + [pltpu.VMEM_SHARED((_N, _D), jnp.float32)]
          + [pltpu.SemaphoreType.DMA] * 10
      ),
  )


def _sc_deg_body(dst_hbm, ones_hbm, zk_hbm, out_hbm,
                 dst0, dst1, dst2, dst3, ones_v, z_v, deg_s,
                 is0, is1, is2, is3, ss0, ss1):
  dsts = [dst0, dst1, dst2, dst3]
  isem, ssem = [is0, is1, is2, is3], [ss0, ss1]
  c = lax.axis_index("c")
  s = lax.axis_index("s")
  wid = c * _NS + s
  cnt = _FPW + jnp.where(wid < _CHUNKS - _FPW * _NW, 1, 0)

  pltpu.sync_copy(zk_hbm, z_v)
  _zero_rows(s, z_v, deg_s)
  pltpu.sync_copy(ones_hbm, ones_v)
  plsc.subcore_barrier()

  def fire_idx(i, b):
    base = _K * wid + (_K * _NW) * i
    pltpu.async_copy(dst_hbm.at[pl.ds(base, _K)], dsts[b], isem[b])

  def wait_idx(b):
    pltpu.make_async_copy(dst_hbm.at[pl.ds(0, _K)], dsts[b], isem[b]).wait()

  def wait_scatter(b):
    pltpu.make_async_copy(ones_v, deg_s.at[pl.ds(0, _K)], ssem[b]).wait()

  fire_idx(0, 0)
  fire_idx(1, 1)
  wait_idx(0)

  def group(it, carry):
    for t in range(4):
      i = 4 * it + t
      r, nr = t % 2, (t + 1) % 2
      nb = (t + 1) % 4

      @pl.when(jnp.logical_and(i >= 1, i + 1 < cnt))
      def _():
        wait_scatter(nr)

      @pl.when(i + 2 < cnt)
      def _():
        fire_idx(i + 2, (t + 2) % 4)

      @pl.when(i + 1 < cnt)
      def _():
        wait_idx(nb)

      @pl.when(i < cnt)
      def _():
        pltpu.async_copy(ones_v, deg_s.at[dsts[t]], ssem[r], add=True)
    return carry

  lax.fori_loop(0, (_FPW + 1 + 3) // 4, group, 0)
  wait_scatter(0)
  wait_scatter(1)

  plsc.subcore_barrier()
  _round_robin(s, lambda r: pltpu.sync_copy(
      deg_s.at[pl.ds(r, _ZR)], out_hbm.at[pl.ds(c * _N + r, _ZR)]))


@functools.cache
def _get_sc_deg():
  mesh = plsc.VectorSubcoreMesh(core_axis_name="c", subcore_axis_name="s",
                                num_cores=_NC, num_subcores=_NS)
  return pl.kernel(
      _sc_deg_body,
      out_type=jax.ShapeDtypeStruct((_NC * _N, _D), jnp.float32),
      mesh=mesh,
      scratch_types=(
          [pltpu.VMEM((_K,), jnp.int32)] * 4
          + [pltpu.VMEM((_K, _D), jnp.float32),
             pltpu.VMEM((_K, _D), jnp.float32),
             pltpu.VMEM_SHARED((_N, _D), jnp.float32)]
          + [pltpu.SemaphoreType.DMA] * 6
      ),
  )

_R = 1000
_G = _N // _R


def _tc1a_body(x_ref, w_ref, h_ref):
  h_ref[...] = jnp.dot(x_ref[...], w_ref[...],
                       preferred_element_type=jnp.float32)


_tc1a = pl.pallas_call(
    _tc1a_body,
    grid=(_G,),
    in_specs=[
        pl.BlockSpec((_R, _D), lambda i: (i, 0)),
        pl.BlockSpec((_D, _D), lambda i: (0, 0)),
    ],
    out_specs=pl.BlockSpec((_R, _D), lambda i: (i, 0)),
    out_shape=jax.ShapeDtypeStruct((_N, _D), jnp.float32),
)


def _tc1b_body(dlo_ref, dhi_ref, h_ref, g_ref, dinv_ref):
  deg = dlo_ref[:, 0:1] + dhi_ref[:, 0:1] + 1.0
  dinv = lax.rsqrt(deg)
  g_ref[...] = h_ref[...] * dinv
  dinv_ref[...] = dinv


_tc1b = pl.pallas_call(
    _tc1b_body,
    grid=(_G,),
    in_specs=[
        pl.BlockSpec((_R, _D), lambda i: (i, 0)),
        pl.BlockSpec((_R, _D), lambda i: (i + _G, 0)),
        pl.BlockSpec((_R, _D), lambda i: (i, 0)),
    ],
    out_specs=[
        pl.BlockSpec((_R, _D), lambda i: (i, 0)),
        pl.BlockSpec((_R, 1), lambda i: (i, 0)),
    ],
    out_shape=[
        jax.ShapeDtypeStruct((_N, _D), jnp.float32),
        jax.ShapeDtypeStruct((_N, 1), jnp.float32),
    ],
)


def _tc2_body(alo_ref, ahi_ref, g_ref, dinv_ref, b_ref, w_ref, o_ref):
  dinv = dinv_ref[...]
  pre = (alo_ref[...] + ahi_ref[...] + g_ref[...]) * dinv + b_ref[...]
  x2 = jnp.maximum(pre, 0.0)
  o_ref[...] = jnp.dot(
      x2, w_ref[...], preferred_element_type=jnp.float32) * dinv


_tc2 = pl.pallas_call(
    _tc2_body,
    grid=(_G,),
    in_specs=[
        pl.BlockSpec((_R, _D), lambda i: (i, 0)),
        pl.BlockSpec((_R, _D), lambda i: (i + _G, 0)),
        pl.BlockSpec((_R, _D), lambda i: (i, 0)),
        pl.BlockSpec((_R, 1), lambda i: (i, 0)),
        pl.BlockSpec((1, _D), lambda i: (0, 0)),
        pl.BlockSpec((_D, _D), lambda i: (0, 0)),
    ],
    out_specs=pl.BlockSpec((_R, _D), lambda i: (i, 0)),
    out_shape=jax.ShapeDtypeStruct((_N, _D), jnp.float32),
)

_C = 40


def _tc3_body(alo_ref, ahi_ref, g_ref, dinv_ref, b_ref, wl_ref, bl_ref, o_ref):
  pre = (alo_ref[...] + ahi_ref[...] + g_ref[...]) * dinv_ref[...] + b_ref[...]
  x3 = jnp.maximum(pre, 0.0)
  o_ref[...] = jnp.dot(
      x3, wl_ref[...], preferred_element_type=jnp.float32) + bl_ref[...]


_tc3 = pl.pallas_call(
    _tc3_body,
    grid=(_G,),
    in_specs=[
        pl.BlockSpec((_R, _D), lambda i: (i, 0)),
        pl.BlockSpec((_R, _D), lambda i: (i + _G, 0)),
        pl.BlockSpec((_R, _D), lambda i: (i, 0)),
        pl.BlockSpec((_R, 1), lambda i: (i, 0)),
        pl.BlockSpec((1, _D), lambda i: (0, 0)),
        pl.BlockSpec((_D, _C), lambda i: (0, 0)),
        pl.BlockSpec((1, _C), lambda i: (0, 0)),
    ],
    out_specs=pl.BlockSpec((_R, _C), lambda i: (i, 0)),
    out_shape=jax.ShapeDtypeStruct((_N, _C), jnp.float32),
)


def kernel(x, edge_index, W1, b1, W2, b2, Wlin, blin):
  src = edge_index[0]
  dst = edge_index[1]
  zk = jnp.zeros((_K, _D), jnp.float32)
  ones = jnp.ones((_K, _D), jnp.float32)

  sc_deg = _get_sc_deg()
  sc_scatter = _get_sc_scatter()
  degp = sc_deg(dst, ones, zk)
  h1 = _tc1a(x, W1)
  g1, dinv = _tc1b(degp, degp, h1)
  a1 = sc_scatter(g1, src, dst, zk)
  g2 = _tc2(a1, a1, g1, dinv, b1.reshape(1, _D), W2)
  a2 = sc_scatter(g2, src, dst, zk)
  return _tc3(a2, a2, g2, dinv, b2.reshape(1, _D), Wlin, blin.reshape(1, _C))


# speedup vs baseline: 34.4056x; 1.1991x over previous
import functools

import jax
import jax.numpy as jnp
from jax import lax
from jax.experimental import pallas as pl
from jax.experimental.pallas import tpu as pltpu
from jax.experimental.pallas import tpu_sc as plsc

_N = 10000
_E = 320000
_D = 128
_NC = 2
_NS = 16
_NW = _NC * _NS
_K = 128
_CHUNKS = _E // _K
_FPW = _CHUNKS // _NW
_NB = 3
_ZR = 200
_NZ = _N // _ZR

def _round_robin(s, body):
  for j in range((_NZ + _NS - 1) // _NS):
    zi = s + _NS * j

    @pl.when(zi < _NZ)
    def _():
      body(zi * _ZR)


_ZF = _N // _K


def _zero_rows(s, zsrc, dstref):
  for j in range((_ZF + _NS - 1) // _NS):
    zi = s + _NS * j

    @pl.when(zi < _ZF)
    def _():
      pltpu.sync_copy(zsrc, dstref.at[pl.ds(zi * _K, _K)])

  @pl.when(s == 0)
  def _():
    pltpu.sync_copy(zsrc.at[pl.ds(0, _N - _ZF * _K)],
                    dstref.at[pl.ds(_ZF * _K, _N - _ZF * _K)])


def _sc_scatter_body(g_hbm, src_hbm, dst_hbm, zk_hbm, out_hbm,
                     src0, src1, src2, src3, dst0, dst1, dst2, dst3,
                     row0, row1, row2, agg_s,
                     is0, is1, is2, is3, gs0, gs1, gs2, ss0, ss1, ss2):
  srcs, dsts = [src0, src1, src2, src3], [dst0, dst1, dst2, dst3]
  rows = [row0, row1, row2]
  isem, gsem, ssem = [is0, is1, is2, is3], [gs0, gs1, gs2], [ss0, ss1, ss2]
  c = lax.axis_index("c")
  s = lax.axis_index("s")
  wid = c * _NS + s
  cnt = _FPW + jnp.where(wid < _CHUNKS - _FPW * _NW, 1, 0)

  pltpu.sync_copy(zk_hbm, row0)
  _zero_rows(s, row0, agg_s)
  plsc.subcore_barrier()

  def fire_idx(i, b):
    base = _K * wid + (_K * _NW) * i
    pltpu.async_copy(src_hbm.at[pl.ds(base, _K)], srcs[b], isem[b])
    pltpu.async_copy(dst_hbm.at[pl.ds(base, _K)], dsts[b], isem[b])

  def wait_idx(b):
    pltpu.make_async_copy(src_hbm.at[pl.ds(0, _K)], srcs[b], isem[b]).wait()
    pltpu.make_async_copy(dst_hbm.at[pl.ds(0, _K)], dsts[b], isem[b]).wait()

  def wait_scatter(b):
    pltpu.make_async_copy(rows[b], agg_s.at[pl.ds(0, _K)], ssem[b]).wait()

  fire_idx(0, 0)
  fire_idx(1, 1)
  wait_idx(0)
  pltpu.async_copy(g_hbm.at[srcs[0]], rows[0], gsem[0])

  def group(it, carry):
    for t in range(12):
      i = 12 * it + t
      r, nr = t % 3, (t + 1) % 3
      nb = (t + 1) % 4

      @pl.when(jnp.logical_and(i >= 2, i + 1 < cnt))
      def _():
        wait_scatter(nr)

      @pl.when(i + 2 < cnt)
      def _():
        fire_idx(i + 2, (t + 2) % 4)

      @pl.when(i + 1 < cnt)
      def _():
        wait_idx(nb)
        pltpu.async_copy(g_hbm.at[srcs[nb]], rows[nr], gsem[nr])

      @pl.when(i < cnt)
      def _():
        pltpu.make_async_copy(g_hbm.at[pl.ds(0, _K)], rows[r], gsem[r]).wait()
        pltpu.async_copy(rows[r], agg_s.at[dsts[t % 4]], ssem[r], add=True)
    return carry

  lax.fori_loop(0, (_FPW + 1 + 11) // 12, group, 0)
  wait_scatter(0)
  wait_scatter(1)
  wait_scatter(2)

  plsc.subcore_barrier()
  _round_robin(s, lambda r: pltpu.sync_copy(
      agg_s.at[pl.ds(r, _ZR)], out_hbm.at[pl.ds(c * _N + r, _ZR)]))


@functools.cache
def _get_sc_scatter():
  mesh = plsc.VectorSubcoreMesh(core_axis_name="c", subcore_axis_name="s",
                                num_cores=_NC, num_subcores=_NS)
  return pl.kernel(
      _sc_scatter_body,
      out_type=jax.ShapeDtypeStruct((_NC * _N, _D), jnp.float32),
      mesh=mesh,
      scratch_types=(
          [pltpu.VMEM((_K,), jnp.int32)] * 8
          + [pltpu.VMEM((_K, _D), jnp.float32)] * 3
          + [pltpu.VMEM_SHARED((_N, _D), jnp.float32)]
          + [pltpu.SemaphoreType.DMA] * 10
      ),
  )


_EPW = _E // _NW
_HR = 80
_SEG = _HR // _NS


def _sc_deg_body(dst_hbm, out_hbm, dst_v, acc2, seg_v, outb, share_s):
  c = lax.axis_index("c")
  s = lax.axis_index("s")
  wid = c * _NS + s
  zero = jnp.zeros((16,), jnp.float32)

  def z(j, carry):
    for cc in range(8):
      acc2[j, pl.ds(cc * 16, 16)] = zero
    return carry

  lax.fori_loop(0, _HR, z, 0)
  pltpu.sync_copy(dst_hbm.at[pl.ds(wid * _EPW, _EPW)], dst_v)
  ones = jnp.ones((16,), jnp.float32)

  def step(j, carry):
    for cc in range(8):
      idx = dst_v[pl.ds(j * 128 + cc * 16, 16)]
      plsc.addupdate_scatter(acc2, [lax.shift_right_logical(idx, 7),
                                    lax.bitwise_and(idx, 127)], ones)
    return carry

  lax.fori_loop(0, _EPW // 128, step, 0)
  idx = dst_v[pl.ds(_EPW - 16, 16)]
  plsc.addupdate_scatter(acc2, [lax.shift_right_logical(idx, 7),
                                lax.bitwise_and(idx, 127)], ones)

  for ss in range(_NS):
    @pl.when(s == ss)
    def _():
      pltpu.sync_copy(acc2, share_s.at[pl.ds(ss * _HR, _HR)])

  plsc.subcore_barrier()
  for ss in range(_NS):
    @pl.when(s == ss)
    def _():
      for k in range(_NS):
        pltpu.sync_copy(share_s.at[pl.ds(k * _HR + _SEG * ss, _SEG)],
                        seg_v.at[pl.ds(k * _SEG, _SEG)])

  for r in range(_SEG):
    for cc in range(8):
      t = seg_v[r, pl.ds(cc * 16, 16)]
      for k in range(1, _NS):
        t = t + seg_v[k * _SEG + r, pl.ds(cc * 16, 16)]
      outb[r, pl.ds(cc * 16, 16)] = t
  pltpu.sync_copy(outb, out_hbm.at[wid])


@functools.cache
def _get_sc_deg():
  mesh = plsc.VectorSubcoreMesh(core_axis_name="c", subcore_axis_name="s",
                                num_cores=_NC, num_subcores=_NS)
  return pl.kernel(
      _sc_deg_body,
      out_type=jax.ShapeDtypeStruct((_NC * _NS, _SEG, 128), jnp.float32),
      mesh=mesh,
      compiler_params=pltpu.CompilerParams(needs_layout_passes=False),
      scratch_types=[
          pltpu.VMEM((_EPW,), jnp.int32),
          pltpu.VMEM((_HR, 128), jnp.float32),
          pltpu.VMEM((_NS * _SEG, 128), jnp.float32),
          pltpu.VMEM((_SEG, 128), jnp.float32),
          pltpu.VMEM_SHARED((_NS * _HR, 128), jnp.float32),
      ],
  )

_R = 1000
_G = _N // _R


def _tc1a_body(x_ref, w_ref, h_ref):
  h_ref[...] = jnp.dot(x_ref[...], w_ref[...],
                       preferred_element_type=jnp.float32)


_tc1a = pl.pallas_call(
    _tc1a_body,
    grid=(_G,),
    in_specs=[
        pl.BlockSpec((_R, _D), lambda i: (i, 0)),
        pl.BlockSpec((_D, _D), lambda i: (0, 0)),
    ],
    out_specs=pl.BlockSpec((_R, _D), lambda i: (i, 0)),
    out_shape=jax.ShapeDtypeStruct((_N, _D), jnp.float32),
)


def _tc1b_body(dlo_ref, dhi_ref, h_ref, g_ref, dinv_ref):
  deg = dlo_ref[...] + dhi_ref[...] + 1.0
  dinv = lax.rsqrt(deg)
  g_ref[...] = h_ref[...] * dinv
  dinv_ref[...] = dinv


_tc1b = pl.pallas_call(
    _tc1b_body,
    grid=(_G,),
    in_specs=[
        pl.BlockSpec((_R, 1), lambda i: (i, 0)),
        pl.BlockSpec((_R, 1), lambda i: (i, 0)),
        pl.BlockSpec((_R, _D), lambda i: (i, 0)),
    ],
    out_specs=[
        pl.BlockSpec((_R, _D), lambda i: (i, 0)),
        pl.BlockSpec((_R, 1), lambda i: (i, 0)),
    ],
    out_shape=[
        jax.ShapeDtypeStruct((_N, _D), jnp.float32),
        jax.ShapeDtypeStruct((_N, 1), jnp.float32),
    ],
)


def _tc2_body(alo_ref, ahi_ref, g_ref, dinv_ref, b_ref, w_ref, o_ref):
  dinv = dinv_ref[...]
  pre = (alo_ref[...] + ahi_ref[...] + g_ref[...]) * dinv + b_ref[...]
  x2 = jnp.maximum(pre, 0.0)
  o_ref[...] = jnp.dot(
      x2, w_ref[...], preferred_element_type=jnp.float32) * dinv


_tc2 = pl.pallas_call(
    _tc2_body,
    grid=(_G,),
    in_specs=[
        pl.BlockSpec((_R, _D), lambda i: (i, 0)),
        pl.BlockSpec((_R, _D), lambda i: (i + _G, 0)),
        pl.BlockSpec((_R, _D), lambda i: (i, 0)),
        pl.BlockSpec((_R, 1), lambda i: (i, 0)),
        pl.BlockSpec((1, _D), lambda i: (0, 0)),
        pl.BlockSpec((_D, _D), lambda i: (0, 0)),
    ],
    out_specs=pl.BlockSpec((_R, _D), lambda i: (i, 0)),
    out_shape=jax.ShapeDtypeStruct((_N, _D), jnp.float32),
)

_C = 40


def _tc3_body(alo_ref, ahi_ref, g_ref, dinv_ref, b_ref, wl_ref, bl_ref, o_ref):
  pre = (alo_ref[...] + ahi_ref[...] + g_ref[...]) * dinv_ref[...] + b_ref[...]
  x3 = jnp.maximum(pre, 0.0)
  o_ref[...] = jnp.dot(
      x3, wl_ref[...], preferred_element_type=jnp.float32) + bl_ref[...]


_tc3 = pl.pallas_call(
    _tc3_body,
    grid=(_G,),
    in_specs=[
        pl.BlockSpec((_R, _D), lambda i: (i, 0)),
        pl.BlockSpec((_R, _D), lambda i: (i + _G, 0)),
        pl.BlockSpec((_R, _D), lambda i: (i, 0)),
        pl.BlockSpec((_R, 1), lambda i: (i, 0)),
        pl.BlockSpec((1, _D), lambda i: (0, 0)),
        pl.BlockSpec((_D, _C), lambda i: (0, 0)),
        pl.BlockSpec((1, _C), lambda i: (0, 0)),
    ],
    out_specs=pl.BlockSpec((_R, _C), lambda i: (i, 0)),
    out_shape=jax.ShapeDtypeStruct((_N, _C), jnp.float32),
)


def kernel(x, edge_index, W1, b1, W2, b2, Wlin, blin):
  src = edge_index[0]
  dst = edge_index[1]
  zk = jnp.zeros((_K, _D), jnp.float32)

  sc_deg = _get_sc_deg()
  sc_scatter = _get_sc_scatter()
  degp = sc_deg(dst)
  dflat = degp.reshape(_NC, _NS * _SEG * 128)
  dlo = dflat[0, :_N].reshape(_N, 1)
  dhi = dflat[1, :_N].reshape(_N, 1)
  h1 = _tc1a(x, W1)
  g1, dinv = _tc1b(dlo, dhi, h1)
  a1 = sc_scatter(g1, src, dst, zk)
  g2 = _tc2(a1, a1, g1, dinv, b1.reshape(1, _D), W2)
  a2 = sc_scatter(g2, src, dst, zk)
  return _tc3(a2, a2, g2, dinv, b2.reshape(1, _D), Wlin, blin.reshape(1, _C))
